# baseline (speedup 1.0000x reference)
import functools

import jax
import jax.numpy as jnp
from jax import lax
from jax.experimental import pallas as pl
from jax.experimental.pallas import tpu as pltpu
from jax.experimental.pallas import tpu_sc as plsc

EMB = 64
EMB_PAD = 128
B = 1024
C_BLK = 8
H = 32
W = 32
H_BLK = 8
NUM_CORES = 2
NUM_SUBCORES = 16
NUM_WORKERS = NUM_CORES * NUM_SUBCORES
B_PER_W = B // NUM_WORKERS


def _gather_sc(labels_hbm, table_hbm, x_hbm, idx_v, rows_v, sem):
    wid = lax.axis_index("s") * NUM_CORES + lax.axis_index("c")
    base = wid * B_PER_W
    pltpu.sync_copy(labels_hbm.at[pl.ds(base, B_PER_W)], idx_v)
    pltpu.async_copy(table_hbm.at[idx_v], rows_v, sem).wait()
    pltpu.sync_copy(rows_v, x_hbm.at[pl.ds(base, B_PER_W)])


def _broadcast_tc(x_ref, out_ref, xT_ref):
    i = pl.program_id(0)
    j = pl.program_id(1)

    @pl.when((i == 0) & (j == 0))
    def _transpose():
        xT_ref[...] = x_ref[...].T[:EMB, :]

    xs = xT_ref[pl.ds(i * C_BLK, C_BLK), :]
    out_ref[...] = jnp.broadcast_to(
        xs[:, None, None, :], (C_BLK, H_BLK, W, B)
    )


def kernel(class_labels, embedding_table):
    labels = class_labels.astype(jnp.int32)
    table_pad = jnp.pad(
        embedding_table, ((0, 0), (0, EMB_PAD - EMB))
    )

    mesh = plsc.VectorSubcoreMesh(core_axis_name="c", subcore_axis_name="s")
    gather = functools.partial(
        pl.kernel,
        mesh=mesh,
        out_type=jax.ShapeDtypeStruct((B, EMB_PAD), jnp.float32),
        scratch_types=[
            pltpu.VMEM((B_PER_W,), jnp.int32),
            pltpu.VMEM((B_PER_W, EMB_PAD), jnp.float32),
            pltpu.SemaphoreType.DMA,
        ],
    )(_gather_sc)
    x = gather(labels, table_pad)

    out = pl.pallas_call(
        _broadcast_tc,
        grid=(EMB // C_BLK, H // H_BLK),
        in_specs=[pl.BlockSpec((B, EMB_PAD), lambda i, j: (0, 0))],
        out_specs=pl.BlockSpec((C_BLK, H_BLK, W, B), lambda i, j: (i, j, 0, 0)),
        out_shape=jax.ShapeDtypeStruct((EMB, H, W, B), jnp.float32),
        scratch_shapes=[pltpu.VMEM((EMB, B), jnp.float32)],
    )(x)
    return jnp.transpose(out, (3, 0, 1, 2))


# speedup vs baseline: 1.1937x; 1.1937x over previous
import jax
import jax.numpy as jnp
from jax.experimental import pallas as pl
from jax.experimental.pallas import tpu as pltpu

K_PAD = 1024
EMB = 64
B = 1024
C_BLK = 8
H = 32
W = 32
H_BLK = 16


def _scc_kernel(labels_ref, tableT_ref, out_ref, xT_ref):
    i = pl.program_id(0)
    j = pl.program_id(1)

    @pl.when((i == 0) & (j == 0))
    def _gather():
        labels = labels_ref[...]
        iota = jax.lax.broadcasted_iota(jnp.int32, (K_PAD, B), 0)
        onehotT = (iota == labels).astype(jnp.float32)
        xT_ref[...] = jnp.dot(
            tableT_ref[...], onehotT, preferred_element_type=jnp.float32
        )

    xs = xT_ref[pl.ds(i * C_BLK, C_BLK), :]
    out_ref[...] = jnp.broadcast_to(
        xs[:, None, None, :], (C_BLK, H_BLK, W, B)
    )


def kernel(class_labels, embedding_table):
    labels_row = class_labels.astype(jnp.int32).reshape(1, B)
    tableT = jnp.pad(
        embedding_table.T, ((0, 0), (0, K_PAD - embedding_table.shape[0]))
    )
    out = pl.pallas_call(
        _scc_kernel,
        grid=(EMB // C_BLK, H // H_BLK),
        in_specs=[
            pl.BlockSpec((1, B), lambda i, j: (0, 0)),
            pl.BlockSpec((EMB, K_PAD), lambda i, j: (0, 0)),
        ],
        out_specs=pl.BlockSpec((C_BLK, H_BLK, W, B), lambda i, j: (i, j, 0, 0)),
        out_shape=jax.ShapeDtypeStruct((EMB, H, W, B), jnp.float32),
        scratch_shapes=[pltpu.VMEM((EMB, B), jnp.float32)],
    )(labels_row, tableT)
    return jnp.transpose(out, (3, 0, 1, 2))


# speedup vs baseline: 1.2538x; 1.0504x over previous
import jax
import jax.numpy as jnp
from jax.experimental import pallas as pl
from jax.experimental.pallas import tpu as pltpu

K_PAD = 1024
EMB = 64
B = 1024
C_BLK = 8
H = 32
W = 32
H_BLK = 4


def _scc_kernel(labels_ref, tableT_ref, out_ref, xT_ref):
    i = pl.program_id(0)
    j = pl.program_id(1)

    @pl.when((i == 0) & (j == 0))
    def _gather():
        labels = labels_ref[...]
        iota = jax.lax.broadcasted_iota(jnp.int32, (K_PAD, B), 0)
        onehotT = (iota == labels).astype(jnp.float32)
        xT_ref[...] = jnp.dot(
            tableT_ref[...], onehotT, preferred_element_type=jnp.float32
        )

    xs = xT_ref[pl.ds(i * C_BLK, C_BLK), :]
    out_ref[...] = jnp.broadcast_to(
        xs[:, None, None, :], (C_BLK, H_BLK, W, B)
    )


def kernel(class_labels, embedding_table):
    labels_row = class_labels.astype(jnp.int32).reshape(1, B)
    tableT = jnp.pad(
        embedding_table.T, ((0, 0), (0, K_PAD - embedding_table.shape[0]))
    )
    out = pl.pallas_call(
        _scc_kernel,
        grid=(EMB // C_BLK, H // H_BLK),
        in_specs=[
            pl.BlockSpec((1, B), lambda i, j: (0, 0)),
            pl.BlockSpec((EMB, K_PAD), lambda i, j: (0, 0)),
        ],
        out_specs=pl.BlockSpec((C_BLK, H_BLK, W, B), lambda i, j: (i, j, 0, 0)),
        out_shape=jax.ShapeDtypeStruct((EMB, H, W, B), jnp.float32),
        scratch_shapes=[pltpu.VMEM((EMB, B), jnp.float32)],
    )(labels_row, tableT)
    return jnp.transpose(out, (3, 0, 1, 2))
